# Initial kernel scaffold; baseline (speedup 1.0000x reference)
import functools

import jax
import jax.numpy as jnp
from jax import lax
from jax.experimental import pallas as pl
from jax.experimental.pallas import tpu as pltpu
from jax.experimental.pallas import tpu_sc as plsc

N_NODES = 10000
NNZ = 320000
F = 128

NC = 2
NS = 16
NW = NC * NS
PER_TILE = NNZ // NW
CHUNK = 80
NCHUNKS = PER_TILE // CHUNK
ROWS_PER_TILE = N_NODES // NS

_mesh = plsc.VectorSubcoreMesh(core_axis_name="c", subcore_axis_name="s")


@functools.partial(
    pl.kernel,
    mesh=_mesh,
    out_type=jax.ShapeDtypeStruct((NC, N_NODES, F), jnp.float32),
    scratch_types=[
        pltpu.VMEM_SHARED((N_NODES, F), jnp.float32),
        pltpu.VMEM((CHUNK,), jnp.int32),
        pltpu.VMEM((CHUNK,), jnp.int32),
        pltpu.VMEM((CHUNK, F), jnp.float32),
        pltpu.SemaphoreType.DMA,
    ],
)
def _seg_sum_sc(table_hbm, sidx_hbm, didx_hbm, zeros_hbm, out_hbm,
                acc_sh, si_v, di_v, rows_v, sem):
    c = lax.axis_index("c")
    s = lax.axis_index("s")
    wid = c * NS + s
    base = wid * PER_TILE
    row0 = s * ROWS_PER_TILE

    pltpu.sync_copy(zeros_hbm.at[pl.ds(row0, ROWS_PER_TILE)],
                    acc_sh.at[pl.ds(row0, ROWS_PER_TILE)])
    plsc.subcore_barrier()

    @pl.loop(0, NCHUNKS)
    def _(i):
        off = base + i * CHUNK
        pltpu.sync_copy(sidx_hbm.at[pl.ds(off, CHUNK)], si_v)
        pltpu.sync_copy(didx_hbm.at[pl.ds(off, CHUNK)], di_v)
        pltpu.async_copy(table_hbm.at[si_v], rows_v, sem).wait()
        pltpu.sync_copy(rows_v, acc_sh.at[di_v], add=True)

    plsc.subcore_barrier()
    pltpu.sync_copy(acc_sh.at[pl.ds(row0, ROWS_PER_TILE)],
                    out_hbm.at[c, pl.ds(row0, ROWS_PER_TILE)])


@functools.partial(
    pl.kernel,
    mesh=_mesh,
    out_type=jax.ShapeDtypeStruct((NC, 2, N_NODES, 16), jnp.float32),
    scratch_types=[
        pltpu.VMEM_SHARED((N_NODES, 16), jnp.float32),
        pltpu.VMEM_SHARED((N_NODES, 16), jnp.float32),
        pltpu.VMEM((CHUNK,), jnp.int32),
        pltpu.VMEM((CHUNK,), jnp.int32),
        pltpu.VMEM((CHUNK, 16), jnp.float32),
    ],
)
def _count_sc(nidx_hbm, eidx_hbm, zeros16_hbm, out_hbm,
              dacc_sh, bacc_sh, ni_v, ei_v, ones_v):
    c = lax.axis_index("c")
    s = lax.axis_index("s")
    wid = c * NS + s
    base = wid * PER_TILE
    row0 = s * ROWS_PER_TILE

    @pl.loop(0, CHUNK)
    def _(i):
        ones_v[i, :] = jnp.ones((16,), jnp.float32)

    pltpu.sync_copy(zeros16_hbm.at[pl.ds(row0, ROWS_PER_TILE)],
                    dacc_sh.at[pl.ds(row0, ROWS_PER_TILE)])
    pltpu.sync_copy(zeros16_hbm.at[pl.ds(row0, ROWS_PER_TILE)],
                    bacc_sh.at[pl.ds(row0, ROWS_PER_TILE)])
    plsc.subcore_barrier()

    @pl.loop(0, NCHUNKS)
    def _(i):
        off = base + i * CHUNK
        pltpu.sync_copy(nidx_hbm.at[pl.ds(off, CHUNK)], ni_v)
        pltpu.sync_copy(eidx_hbm.at[pl.ds(off, CHUNK)], ei_v)
        pltpu.sync_copy(ones_v, dacc_sh.at[ni_v], add=True)
        pltpu.sync_copy(ones_v, bacc_sh.at[ei_v], add=True)

    plsc.subcore_barrier()
    pltpu.sync_copy(dacc_sh.at[pl.ds(row0, ROWS_PER_TILE)],
                    out_hbm.at[c, 0, pl.ds(row0, ROWS_PER_TILE)])
    pltpu.sync_copy(bacc_sh.at[pl.ds(row0, ROWS_PER_TILE)],
                    out_hbm.at[c, 1, pl.ds(row0, ROWS_PER_TILE)])


_MM_ROWS = 400


def _mm_body(x_ref, w_ref, o_ref):
    o_ref[...] = jnp.dot(x_ref[...], w_ref[...],
                         preferred_element_type=jnp.float32)


def _matmul(x, w):
    return pl.pallas_call(
        _mm_body,
        grid=(N_NODES // _MM_ROWS,),
        in_specs=[
            pl.BlockSpec((_MM_ROWS, F), lambda i: (i, 0)),
            pl.BlockSpec((F, F), lambda i: (0, 0)),
        ],
        out_specs=pl.BlockSpec((_MM_ROWS, F), lambda i: (i, 0)),
        out_shape=jax.ShapeDtypeStruct((N_NODES, F), jnp.float32),
    )(x, w)


_CB_ROWS = 2000


def _combine_body(relu, p_ref, cnt_ref, bias_ref, o_ref):
    cn = cnt_ref[0] + cnt_ref[1]
    sv = cn[:, 0:1]
    scale = jnp.where(sv > 0, 1.0 / sv, 0.0)
    o = (p_ref[0] + p_ref[1]) * scale + bias_ref[...]
    if relu:
        o = jnp.maximum(o, 0.0)
    o_ref[...] = o


def _combine(partials, cnt, bias_row, relu):
    return pl.pallas_call(
        functools.partial(_combine_body, relu),
        grid=(N_NODES // _CB_ROWS,),
        in_specs=[
            pl.BlockSpec((NC, _CB_ROWS, F), lambda i: (0, i, 0)),
            pl.BlockSpec((NC, _CB_ROWS, 16), lambda i: (0, i, 0)),
            pl.BlockSpec((1, F), lambda i: (0, 0)),
        ],
        out_specs=pl.BlockSpec((_CB_ROWS, F), lambda i: (i, 0)),
        out_shape=jax.ShapeDtypeStruct((N_NODES, F), jnp.float32),
    )(partials, cnt, bias_row)


def kernel(x, hyperedge_index, W1, b1, W2, b2):
    node_idx = hyperedge_index[0]
    edge_idx = hyperedge_index[1]
    zeros128 = jnp.zeros((N_NODES, F), jnp.float32)
    zeros16 = jnp.zeros((N_NODES, 16), jnp.float32)
    b1r = b1.reshape(1, F)
    b2r = b2.reshape(1, F)
    zb = jnp.zeros((1, F), jnp.float32)

    cnt = _count_sc(node_idx, edge_idx, zeros16)
    dcnt = cnt[:, 0]
    bcnt = cnt[:, 1]

    xw1 = _matmul(x, W1)
    ep = _seg_sum_sc(xw1, node_idx, edge_idx, zeros128)
    e = _combine(ep, bcnt, zb, relu=False)
    op = _seg_sum_sc(e, edge_idx, node_idx, zeros128)
    h = _combine(op, dcnt, b1r, relu=True)

    xw2 = _matmul(h, W2)
    ep2 = _seg_sum_sc(xw2, node_idx, edge_idx, zeros128)
    e2 = _combine(ep2, bcnt, zb, relu=False)
    op2 = _seg_sum_sc(e2, edge_idx, node_idx, zeros128)
    y = _combine(op2, dcnt, b2r, relu=False)
    return y


# speedup vs baseline: 7.2222x; 7.2222x over previous
import functools

import jax
import jax.numpy as jnp
from jax import lax
from jax.experimental import pallas as pl
from jax.experimental.pallas import tpu as pltpu
from jax.experimental.pallas import tpu_sc as plsc

N_NODES = 10000
N_PAD = 10112
NNZ = 320000
F = 128

NC = 2
NS = 16
NW = NC * NS
PER_TILE = NNZ // NW
CHUNK = 80
NCHUNKS = PER_TILE // CHUNK
ROWS_PER_TILE = N_PAD // NS

_mesh = plsc.VectorSubcoreMesh(core_axis_name="c", subcore_axis_name="s")


@functools.partial(
    pl.kernel,
    mesh=_mesh,
    out_type=jax.ShapeDtypeStruct((NC, N_PAD, F), jnp.float32),
    scratch_types=[
        pltpu.VMEM_SHARED((N_PAD, F), jnp.float32),
        pltpu.VMEM((CHUNK,), jnp.int32),
        pltpu.VMEM((CHUNK,), jnp.int32),
        pltpu.VMEM((CHUNK, F), jnp.float32),
        pltpu.SemaphoreType.DMA,
    ],
)
def _seg_sum_sc(table_hbm, sidx_hbm, didx_hbm, zeros_hbm, out_hbm,
                acc_sh, si_v, di_v, rows_v, sem):
    c = lax.axis_index("c")
    s = lax.axis_index("s")
    wid = c * NS + s
    base = wid * PER_TILE
    row0 = s * ROWS_PER_TILE

    pltpu.sync_copy(zeros_hbm.at[pl.ds(row0, ROWS_PER_TILE)],
                    acc_sh.at[pl.ds(row0, ROWS_PER_TILE)])
    plsc.subcore_barrier()

    @pl.loop(0, NCHUNKS)
    def _(i):
        off = base + i * CHUNK
        pltpu.sync_copy(sidx_hbm.at[pl.ds(off, CHUNK)], si_v)
        pltpu.sync_copy(didx_hbm.at[pl.ds(off, CHUNK)], di_v)
        pltpu.async_copy(table_hbm.at[si_v], rows_v, sem).wait()
        pltpu.sync_copy(rows_v, acc_sh.at[di_v], add=True)

    plsc.subcore_barrier()
    pltpu.sync_copy(acc_sh.at[pl.ds(row0, ROWS_PER_TILE)],
                    out_hbm.at[c, pl.ds(row0, ROWS_PER_TILE)])


@functools.partial(
    pl.kernel,
    mesh=_mesh,
    out_type=jax.ShapeDtypeStruct((NC, N_PAD, F), jnp.float32),
    scratch_types=[
        pltpu.VMEM_SHARED((N_PAD, F), jnp.float32),
        pltpu.VMEM((CHUNK,), jnp.int32),
        pltpu.VMEM((CHUNK, F), jnp.float32),
    ],
)
def _count_sc(didx_hbm, zeros_hbm, ones_hbm, out_hbm, acc_sh, di_v, ones_v):
    c = lax.axis_index("c")
    s = lax.axis_index("s")
    wid = c * NS + s
    base = wid * PER_TILE
    row0 = s * ROWS_PER_TILE

    pltpu.sync_copy(ones_hbm, ones_v)
    pltpu.sync_copy(zeros_hbm.at[pl.ds(row0, ROWS_PER_TILE)],
                    acc_sh.at[pl.ds(row0, ROWS_PER_TILE)])
    plsc.subcore_barrier()

    @pl.loop(0, NCHUNKS)
    def _(i):
        off = base + i * CHUNK
        pltpu.sync_copy(didx_hbm.at[pl.ds(off, CHUNK)], di_v)
        pltpu.sync_copy(ones_v, acc_sh.at[di_v], add=True)

    plsc.subcore_barrier()
    pltpu.sync_copy(acc_sh.at[pl.ds(row0, ROWS_PER_TILE)],
                    out_hbm.at[c, pl.ds(row0, ROWS_PER_TILE)])


_MM_ROWS = 400


def _mm_body(x_ref, w_ref, o_ref):
    o_ref[...] = jnp.dot(x_ref[...], w_ref[...],
                         preferred_element_type=jnp.float32)


def _matmul(x, w):
    return pl.pallas_call(
        _mm_body,
        grid=(N_NODES // _MM_ROWS,),
        in_specs=[
            pl.BlockSpec((_MM_ROWS, F), lambda i: (i, 0)),
            pl.BlockSpec((F, F), lambda i: (0, 0)),
        ],
        out_specs=pl.BlockSpec((_MM_ROWS, F), lambda i: (i, 0)),
        out_shape=jax.ShapeDtypeStruct((N_NODES, F), jnp.float32),
    )(x, w)


_CB_ROWS = 2000


def _combine_body(relu, p_ref, cnt_ref, bias_ref, o_ref):
    cn = cnt_ref[0] + cnt_ref[1]
    sv = cn[:, 0:1]
    scale = jnp.where(sv > 0, 1.0 / sv, 0.0)
    o = (p_ref[0] + p_ref[1]) * scale + bias_ref[...]
    if relu:
        o = jnp.maximum(o, 0.0)
    o_ref[...] = o


def _combine(partials, cnt, bias_row, relu):
    return pl.pallas_call(
        functools.partial(_combine_body, relu),
        grid=(N_NODES // _CB_ROWS,),
        in_specs=[
            pl.BlockSpec((NC, _CB_ROWS, F), lambda i: (0, i, 0)),
            pl.BlockSpec((NC, _CB_ROWS, F), lambda i: (0, i, 0)),
            pl.BlockSpec((1, F), lambda i: (0, 0)),
        ],
        out_specs=pl.BlockSpec((_CB_ROWS, F), lambda i: (i, 0)),
        out_shape=jax.ShapeDtypeStruct((N_NODES, F), jnp.float32),
    )(partials, cnt, bias_row)


def kernel(x, hyperedge_index, W1, b1, W2, b2):
    node_idx = hyperedge_index[0]
    edge_idx = hyperedge_index[1]
    zeros128 = jnp.zeros((N_PAD, F), jnp.float32)
    b1r = b1.reshape(1, F)
    b2r = b2.reshape(1, F)
    zb = jnp.zeros((1, F), jnp.float32)

    ones128 = jnp.ones((CHUNK, F), jnp.float32)
    dcnt = _count_sc(node_idx, zeros128, ones128)
    bcnt = _count_sc(edge_idx, zeros128, ones128)

    xw1 = _matmul(x, W1)
    ep = _seg_sum_sc(xw1, node_idx, edge_idx, zeros128)
    e = _combine(ep, bcnt, zb, relu=False)
    op = _seg_sum_sc(e, edge_idx, node_idx, zeros128)
    h = _combine(op, dcnt, b1r, relu=True)

    xw2 = _matmul(h, W2)
    ep2 = _seg_sum_sc(xw2, node_idx, edge_idx, zeros128)
    e2 = _combine(ep2, bcnt, zb, relu=False)
    op2 = _seg_sum_sc(e2, edge_idx, node_idx, zeros128)
    y = _combine(op2, dcnt, b2r, relu=False)
    return y


# speedup vs baseline: 17.0952x; 2.3671x over previous
import functools

import jax
import jax.numpy as jnp
from jax import lax
from jax.experimental import pallas as pl
from jax.experimental.pallas import tpu as pltpu
from jax.experimental.pallas import tpu_sc as plsc

N_NODES = 10000
N_PAD = 10112
NNZ = 320000
F = 128

NC = 2
NS = 16
NW = NC * NS
PER_TILE = NNZ // NW
CH = 128
NFULL = (PER_TILE // CH) & ~1
TAIL = PER_TILE - NFULL * CH
ROWS_PER_TILE = N_PAD // NS

_mesh = plsc.VectorSubcoreMesh(core_axis_name="c", subcore_axis_name="s")


@functools.partial(
    pl.kernel,
    mesh=_mesh,
    out_type=jax.ShapeDtypeStruct((NC, N_PAD, F), jnp.float32),
    scratch_types=[
        pltpu.VMEM_SHARED((N_PAD, F), jnp.float32),
        pltpu.VMEM((PER_TILE,), jnp.int32),
        pltpu.VMEM((CH,), jnp.int32),
        pltpu.VMEM((CH,), jnp.int32),
        pltpu.VMEM((CH, F), jnp.float32),
        pltpu.VMEM((CH, F), jnp.float32),
        pltpu.VMEM((TAIL,), jnp.int32),
        pltpu.VMEM((TAIL, F), jnp.float32),
        pltpu.SemaphoreType.DMA,
        pltpu.SemaphoreType.DMA,
        pltpu.SemaphoreType.DMA,
        pltpu.SemaphoreType.DMA,
    ],
)
def _seg_sum_sc(table_hbm, sidx_hbm, didx_hbm, zeros_hbm, out_hbm,
                acc_sh, si_all, di0, di1, r0, r1, di_t, r_t,
                semi0, semi1, semg0, semg1):
    c = lax.axis_index("c")
    s = lax.axis_index("s")
    wid = c * NS + s
    base = wid * PER_TILE
    row0 = s * ROWS_PER_TILE

    dis = (di0, di1)
    rs = (r0, r1)
    semis = (semi0, semi1)
    semgs = (semg0, semg1)

    def start(i, b):
        pltpu.async_copy(didx_hbm.at[pl.ds(base + i * CH, CH)], dis[b], semis[b])
        pltpu.async_copy(table_hbm.at[si_all.at[pl.ds(i * CH, CH)]], rs[b], semgs[b])

    def wait(b):
        pltpu.make_async_copy(didx_hbm.at[pl.ds(base, CH)], dis[b], semis[b]).wait()
        pltpu.make_async_copy(zeros_hbm.at[pl.ds(0, CH)], rs[b], semgs[b]).wait()

    pltpu.sync_copy(sidx_hbm.at[pl.ds(base, PER_TILE)], si_all)
    pltpu.sync_copy(zeros_hbm.at[pl.ds(row0, ROWS_PER_TILE)],
                    acc_sh.at[pl.ds(row0, ROWS_PER_TILE)])
    plsc.subcore_barrier()

    start(0, 0)
    start(1, 1)

    @pl.loop(0, NFULL, step=2)
    def _(i):
        wait(0)
        pltpu.sync_copy(rs[0], acc_sh.at[dis[0]], add=True)

        @pl.when(i + 2 < NFULL)
        def _():
            start(i + 2, 0)

        wait(1)
        pltpu.sync_copy(rs[1], acc_sh.at[dis[1]], add=True)

        @pl.when(i + 3 < NFULL)
        def _():
            start(i + 3, 1)

    pltpu.sync_copy(didx_hbm.at[pl.ds(base + NFULL * CH, TAIL)], di_t)
    pltpu.async_copy(table_hbm.at[si_all.at[pl.ds(NFULL * CH, TAIL)]],
                     r_t, semg0).wait()
    pltpu.sync_copy(r_t, acc_sh.at[di_t], add=True)

    plsc.subcore_barrier()
    pltpu.sync_copy(acc_sh.at[pl.ds(row0, ROWS_PER_TILE)],
                    out_hbm.at[c, pl.ds(row0, ROWS_PER_TILE)])


@functools.partial(
    pl.kernel,
    mesh=_mesh,
    out_type=jax.ShapeDtypeStruct((NC, N_PAD, F), jnp.float32),
    scratch_types=[
        pltpu.VMEM_SHARED((N_PAD, F), jnp.float32),
        pltpu.VMEM((CH, F), jnp.float32),
        pltpu.VMEM((CH,), jnp.int32),
        pltpu.VMEM((CH,), jnp.int32),
        pltpu.VMEM((TAIL,), jnp.int32),
        pltpu.SemaphoreType.DMA,
        pltpu.SemaphoreType.DMA,
    ],
)
def _count_sc(didx_hbm, zeros_hbm, ones_hbm, out_hbm,
              acc_sh, ones_v, di0, di1, di_t, semi0, semi1):
    c = lax.axis_index("c")
    s = lax.axis_index("s")
    wid = c * NS + s
    base = wid * PER_TILE
    row0 = s * ROWS_PER_TILE

    dis = (di0, di1)
    semis = (semi0, semi1)

    def start(i, b):
        pltpu.async_copy(didx_hbm.at[pl.ds(base + i * CH, CH)], dis[b], semis[b])

    def wait(b):
        pltpu.make_async_copy(didx_hbm.at[pl.ds(base, CH)], dis[b], semis[b]).wait()

    pltpu.sync_copy(ones_hbm, ones_v)
    pltpu.sync_copy(zeros_hbm.at[pl.ds(row0, ROWS_PER_TILE)],
                    acc_sh.at[pl.ds(row0, ROWS_PER_TILE)])
    plsc.subcore_barrier()

    start(0, 0)
    start(1, 1)

    @pl.loop(0, NFULL, step=2)
    def _(i):
        wait(0)
        pltpu.sync_copy(ones_v, acc_sh.at[dis[0]], add=True)

        @pl.when(i + 2 < NFULL)
        def _():
            start(i + 2, 0)

        wait(1)
        pltpu.sync_copy(ones_v, acc_sh.at[dis[1]], add=True)

        @pl.when(i + 3 < NFULL)
        def _():
            start(i + 3, 1)

    pltpu.sync_copy(didx_hbm.at[pl.ds(base + NFULL * CH, TAIL)], di_t)
    pltpu.sync_copy(ones_v.at[pl.ds(0, TAIL)], acc_sh.at[di_t], add=True)

    plsc.subcore_barrier()
    pltpu.sync_copy(acc_sh.at[pl.ds(row0, ROWS_PER_TILE)],
                    out_hbm.at[c, pl.ds(row0, ROWS_PER_TILE)])


_MM_ROWS = 400


def _mm_body(x_ref, w_ref, o_ref):
    o_ref[...] = jnp.dot(x_ref[...], w_ref[...],
                         preferred_element_type=jnp.float32)


def _matmul(x, w):
    return pl.pallas_call(
        _mm_body,
        grid=(N_NODES // _MM_ROWS,),
        in_specs=[
            pl.BlockSpec((_MM_ROWS, F), lambda i: (i, 0)),
            pl.BlockSpec((F, F), lambda i: (0, 0)),
        ],
        out_specs=pl.BlockSpec((_MM_ROWS, F), lambda i: (i, 0)),
        out_shape=jax.ShapeDtypeStruct((N_NODES, F), jnp.float32),
    )(x, w)


_CB_ROWS = 2000


def _combine_body(relu, p_ref, cnt_ref, bias_ref, o_ref):
    cn = cnt_ref[0] + cnt_ref[1]
    sv = cn[:, 0:1]
    scale = jnp.where(sv > 0, 1.0 / sv, 0.0)
    o = (p_ref[0] + p_ref[1]) * scale + bias_ref[...]
    if relu:
        o = jnp.maximum(o, 0.0)
    o_ref[...] = o


def _combine(partials, cnt, bias_row, relu):
    return pl.pallas_call(
        functools.partial(_combine_body, relu),
        grid=(N_NODES // _CB_ROWS,),
        in_specs=[
            pl.BlockSpec((NC, _CB_ROWS, F), lambda i: (0, i, 0)),
            pl.BlockSpec((NC, _CB_ROWS, F), lambda i: (0, i, 0)),
            pl.BlockSpec((1, F), lambda i: (0, 0)),
        ],
        out_specs=pl.BlockSpec((_CB_ROWS, F), lambda i: (i, 0)),
        out_shape=jax.ShapeDtypeStruct((N_NODES, F), jnp.float32),
    )(partials, cnt, bias_row)


def kernel(x, hyperedge_index, W1, b1, W2, b2):
    node_idx = hyperedge_index[0]
    edge_idx = hyperedge_index[1]
    zeros128 = jnp.zeros((N_PAD, F), jnp.float32)
    b1r = b1.reshape(1, F)
    b2r = b2.reshape(1, F)
    zb = jnp.zeros((1, F), jnp.float32)

    ones128 = jnp.ones((CH, F), jnp.float32)
    dcnt = _count_sc(node_idx, zeros128, ones128)
    bcnt = _count_sc(edge_idx, zeros128, ones128)

    xw1 = _matmul(x, W1)
    ep = _seg_sum_sc(xw1, node_idx, edge_idx, zeros128)
    e = _combine(ep, bcnt, zb, relu=False)
    op = _seg_sum_sc(e, edge_idx, node_idx, zeros128)
    h = _combine(op, dcnt, b1r, relu=True)

    xw2 = _matmul(h, W2)
    ep2 = _seg_sum_sc(xw2, node_idx, edge_idx, zeros128)
    e2 = _combine(ep2, bcnt, zb, relu=False)
    op2 = _seg_sum_sc(e2, edge_idx, node_idx, zeros128)
    y = _combine(op2, dcnt, b2r, relu=False)
    return y


# speedup vs baseline: 17.9876x; 1.0522x over previous
import functools

import jax
import jax.numpy as jnp
from jax import lax
from jax.experimental import pallas as pl
from jax.experimental.pallas import tpu as pltpu
from jax.experimental.pallas import tpu_sc as plsc

N_NODES = 10000
N_PAD = 10112
NNZ = 320000
F = 128

NC = 2
NS = 16
NW = NC * NS
PER_TILE = NNZ // NW
CH = 104
NB = 3
NFULL = (PER_TILE // CH) // NB * NB
TAIL = PER_TILE - NFULL * CH
STRIPE = 624
STRIPE_REM = N_NODES - NS * STRIPE

_mesh = plsc.VectorSubcoreMesh(core_axis_name="c", subcore_axis_name="s")


@functools.partial(
    pl.kernel,
    mesh=_mesh,
    out_type=jax.ShapeDtypeStruct((NC, N_NODES, F), jnp.float32),
    scratch_types=[
        pltpu.VMEM_SHARED((N_NODES, F), jnp.float32),
        pltpu.VMEM((PER_TILE,), jnp.int32),
        pltpu.VMEM((CH,), jnp.int32),
        pltpu.VMEM((CH,), jnp.int32),
        pltpu.VMEM((CH,), jnp.int32),
        pltpu.VMEM((CH, F), jnp.float32),
        pltpu.VMEM((CH, F), jnp.float32),
        pltpu.VMEM((CH, F), jnp.float32),
        pltpu.VMEM((TAIL,), jnp.int32),
        pltpu.SemaphoreType.DMA,
        pltpu.SemaphoreType.DMA,
        pltpu.SemaphoreType.DMA,
        pltpu.SemaphoreType.DMA,
        pltpu.SemaphoreType.DMA,
        pltpu.SemaphoreType.DMA,
    ],
)
def _seg_sum_sc(table_hbm, sidx_hbm, didx_hbm, zeros_hbm, out_hbm,
                acc_sh, si_all, di0, di1, di2, r0, r1, r2, di_t,
                semi0, semi1, semi2, semg0, semg1, semg2):
    c = lax.axis_index("c")
    s = lax.axis_index("s")
    wid = c * NS + s
    base = wid * PER_TILE
    row0 = s * STRIPE

    dis = (di0, di1, di2)
    rs = (r0, r1, r2)
    semis = (semi0, semi1, semi2)
    semgs = (semg0, semg1, semg2)

    def start(i, b):
        pltpu.async_copy(didx_hbm.at[pl.ds(base + i * CH, CH)], dis[b], semis[b])
        pltpu.async_copy(table_hbm.at[si_all.at[pl.ds(i * CH, CH)]], rs[b], semgs[b])

    def wait(b):
        pltpu.make_async_copy(didx_hbm.at[pl.ds(base, CH)], dis[b], semis[b]).wait()
        pltpu.make_async_copy(zeros_hbm.at[pl.ds(0, CH)], rs[b], semgs[b]).wait()

    pltpu.sync_copy(sidx_hbm.at[pl.ds(base, PER_TILE)], si_all)
    pltpu.sync_copy(zeros_hbm.at[pl.ds(row0, STRIPE)],
                    acc_sh.at[pl.ds(row0, STRIPE)])

    @pl.when(s == 0)
    def _():
        pltpu.sync_copy(zeros_hbm.at[pl.ds(NS * STRIPE, STRIPE_REM)],
                        acc_sh.at[pl.ds(NS * STRIPE, STRIPE_REM)])

    plsc.subcore_barrier()

    for _b in range(NB):
        start(_b, _b)

    @pl.loop(0, NFULL, step=NB)
    def _(i):
        for _b in range(NB):
            wait(_b)
            pltpu.sync_copy(rs[_b], acc_sh.at[dis[_b]], add=True)

            @pl.when(i + _b + NB < NFULL)
            def _(b=_b):
                start(i + b + NB, b)

    pltpu.sync_copy(didx_hbm.at[pl.ds(base + NFULL * CH, TAIL)], di_t)
    pltpu.async_copy(table_hbm.at[si_all.at[pl.ds(NFULL * CH, TAIL)]],
                     r0.at[pl.ds(0, TAIL)], semg0).wait()
    pltpu.sync_copy(r0.at[pl.ds(0, TAIL)], acc_sh.at[di_t], add=True)

    plsc.subcore_barrier()
    pltpu.sync_copy(acc_sh.at[pl.ds(row0, STRIPE)],
                    out_hbm.at[c, pl.ds(row0, STRIPE)])

    @pl.when(s == 0)
    def _():
        pltpu.sync_copy(acc_sh.at[pl.ds(NS * STRIPE, STRIPE_REM)],
                        out_hbm.at[c, pl.ds(NS * STRIPE, STRIPE_REM)])


@functools.partial(
    pl.kernel,
    mesh=_mesh,
    out_type=jax.ShapeDtypeStruct((NC, N_NODES, F), jnp.float32),
    scratch_types=[
        pltpu.VMEM_SHARED((N_NODES, F), jnp.float32),
        pltpu.VMEM((CH, F), jnp.float32),
        pltpu.VMEM((CH,), jnp.int32),
        pltpu.VMEM((CH,), jnp.int32),
        pltpu.VMEM((TAIL,), jnp.int32),
        pltpu.SemaphoreType.DMA,
        pltpu.SemaphoreType.DMA,
    ],
)
def _count_sc(didx_hbm, zeros_hbm, ones_hbm, out_hbm,
              acc_sh, ones_v, di0, di1, di_t, semi0, semi1):
    c = lax.axis_index("c")
    s = lax.axis_index("s")
    wid = c * NS + s
    base = wid * PER_TILE
    row0 = s * STRIPE

    dis = (di0, di1)
    semis = (semi0, semi1)

    def start(i, b):
        pltpu.async_copy(didx_hbm.at[pl.ds(base + i * CH, CH)], dis[b], semis[b])

    def wait(b):
        pltpu.make_async_copy(didx_hbm.at[pl.ds(base, CH)], dis[b], semis[b]).wait()

    pltpu.sync_copy(ones_hbm, ones_v)
    pltpu.sync_copy(zeros_hbm.at[pl.ds(row0, STRIPE)],
                    acc_sh.at[pl.ds(row0, STRIPE)])

    @pl.when(s == 0)
    def _():
        pltpu.sync_copy(zeros_hbm.at[pl.ds(NS * STRIPE, STRIPE_REM)],
                        acc_sh.at[pl.ds(NS * STRIPE, STRIPE_REM)])

    plsc.subcore_barrier()

    start(0, 0)
    start(1, 1)

    @pl.loop(0, NFULL, step=2)
    def _(i):
        wait(0)
        pltpu.sync_copy(ones_v, acc_sh.at[dis[0]], add=True)

        @pl.when(i + 2 < NFULL)
        def _():
            start(i + 2, 0)

        wait(1)
        pltpu.sync_copy(ones_v, acc_sh.at[dis[1]], add=True)

        @pl.when(i + 3 < NFULL)
        def _():
            start(i + 3, 1)

    pltpu.sync_copy(didx_hbm.at[pl.ds(base + NFULL * CH, TAIL)], di_t)
    pltpu.sync_copy(ones_v.at[pl.ds(0, TAIL)], acc_sh.at[di_t], add=True)

    plsc.subcore_barrier()
    pltpu.sync_copy(acc_sh.at[pl.ds(row0, STRIPE)],
                    out_hbm.at[c, pl.ds(row0, STRIPE)])

    @pl.when(s == 0)
    def _():
        pltpu.sync_copy(acc_sh.at[pl.ds(NS * STRIPE, STRIPE_REM)],
                        out_hbm.at[c, pl.ds(NS * STRIPE, STRIPE_REM)])


_MM_ROWS = 400


def _mm_body(x_ref, w_ref, o_ref):
    o_ref[...] = jnp.dot(x_ref[...], w_ref[...],
                         preferred_element_type=jnp.float32)


def _matmul(x, w):
    return pl.pallas_call(
        _mm_body,
        grid=(N_NODES // _MM_ROWS,),
        in_specs=[
            pl.BlockSpec((_MM_ROWS, F), lambda i: (i, 0)),
            pl.BlockSpec((F, F), lambda i: (0, 0)),
        ],
        out_specs=pl.BlockSpec((_MM_ROWS, F), lambda i: (i, 0)),
        out_shape=jax.ShapeDtypeStruct((N_NODES, F), jnp.float32),
    )(x, w)


_CB_ROWS = 2000


def _combine_body(relu, p_ref, cnt_ref, bias_ref, o_ref):
    cn = cnt_ref[0] + cnt_ref[1]
    sv = cn[:, 0:1]
    scale = jnp.where(sv > 0, 1.0 / sv, 0.0)
    o = (p_ref[0] + p_ref[1]) * scale + bias_ref[...]
    if relu:
        o = jnp.maximum(o, 0.0)
    o_ref[...] = o


def _combine(partials, cnt, bias_row, relu):
    return pl.pallas_call(
        functools.partial(_combine_body, relu),
        grid=(N_NODES // _CB_ROWS,),
        in_specs=[
            pl.BlockSpec((NC, _CB_ROWS, F), lambda i: (0, i, 0)),
            pl.BlockSpec((NC, _CB_ROWS, F), lambda i: (0, i, 0)),
            pl.BlockSpec((1, F), lambda i: (0, 0)),
        ],
        out_specs=pl.BlockSpec((_CB_ROWS, F), lambda i: (i, 0)),
        out_shape=jax.ShapeDtypeStruct((N_NODES, F), jnp.float32),
    )(partials, cnt, bias_row)


def kernel(x, hyperedge_index, W1, b1, W2, b2):
    node_idx = hyperedge_index[0]
    edge_idx = hyperedge_index[1]
    zeros128 = jnp.zeros((N_NODES, F), jnp.float32)
    b1r = b1.reshape(1, F)
    b2r = b2.reshape(1, F)
    zb = jnp.zeros((1, F), jnp.float32)

    ones128 = jnp.ones((CH, F), jnp.float32)
    dcnt = _count_sc(node_idx, zeros128, ones128)
    bcnt = _count_sc(edge_idx, zeros128, ones128)

    xw1 = _matmul(x, W1)
    ep = _seg_sum_sc(xw1, node_idx, edge_idx, zeros128)
    e = _combine(ep, bcnt, zb, relu=False)
    op = _seg_sum_sc(e, edge_idx, node_idx, zeros128)
    h = _combine(op, dcnt, b1r, relu=True)

    xw2 = _matmul(h, W2)
    ep2 = _seg_sum_sc(xw2, node_idx, edge_idx, zeros128)
    e2 = _combine(ep2, bcnt, zb, relu=False)
    op2 = _seg_sum_sc(e2, edge_idx, node_idx, zeros128)
    y = _combine(op2, dcnt, b2r, relu=False)
    return y
